# baseline (speedup 1.0000x reference)
import functools

import jax
import jax.numpy as jnp
from jax import lax
from jax.experimental import pallas as pl
from jax.experimental.pallas import tpu as pltpu
from jax.experimental.pallas import tpu_sc as plsc

D_MODEL = 64
NUM_CORES = 2
NUM_SUBCORES = 16
NW = NUM_CORES * NUM_SUBCORES
NBUF = 4


@functools.partial(jax.jit, static_argnames=("bsz", "seq"))
def _emb_lookup(table, x, bsz, seq):
    mesh = plsc.VectorSubcoreMesh(
        core_axis_name="c", subcore_axis_name="s",
        num_cores=NUM_CORES, num_subcores=NUM_SUBCORES)
    rows_per_w = bsz // NW

    @functools.partial(
        pl.kernel,
        out_type=jax.ShapeDtypeStruct((bsz, seq, D_MODEL), jnp.float32),
        mesh=mesh,
        scratch_types=[
            pltpu.VMEM((rows_per_w, seq), jnp.int32),
            pltpu.VMEM((NBUF, seq, D_MODEL), jnp.float32),
            pltpu.SemaphoreType.DMA,
            pltpu.SemaphoreType.DMA((NBUF,)),
            pltpu.SemaphoreType.DMA((NBUF,)),
        ],
        compiler_params=pltpu.CompilerParams(use_tc_tiling_on_sc=False),
    )
    def k(table_hbm, x_hbm, out_hbm, idx_v, rows_v, isem, gsems, osems):
        wid = lax.axis_index("s") * NUM_CORES + lax.axis_index("c")
        base = wid * rows_per_w

        cp = pltpu.make_async_copy(
            x_hbm.at[pl.ds(base, rows_per_w)], idx_v, isem)
        cp.start()
        cp.wait()

        def g_copy(j, b):
            return pltpu.make_async_copy(
                table_hbm.at[idx_v.at[j]], rows_v.at[b], gsems.at[b])

        def o_copy(j, b):
            return pltpu.make_async_copy(
                rows_v.at[b], out_hbm.at[base + j], osems.at[b])

        for b in range(NBUF):
            g_copy(b, b).start()

        n_rounds = rows_per_w // NBUF

        def round_body(r, carry):
            for b in range(NBUF):
                j = r * NBUF + b
                g_copy(j, b).wait()
                o_copy(j, b).start()
            for b in range(NBUF):
                j = r * NBUF + b
                o_copy(j, b).wait()
                jn = j + NBUF

                @pl.when(jn < rows_per_w)
                def _():
                    g_copy(jn, b).start()

            return carry

        lax.fori_loop(0, n_rounds, round_body, 0)

    return k(table, x)


def kernel(x, table):
    bsz, seq = x.shape
    return _emb_lookup(table, x, bsz, seq)
